# baseline (speedup 1.0000x reference)
import functools

import jax
import jax.numpy as jnp
from jax import lax
from jax.experimental import pallas as pl
from jax.experimental.pallas import tpu as pltpu
from jax.experimental.pallas import tpu_sc as plsc

N_NODES = 10000
D = 128
H = 64
N_EDGES = 320000
NPAD = 10240
ROWS_PER_TILE = NPAD // 16
K1_EDGES_PER_TILE = N_EDGES // 16
K1_CHUNK = 80
K1_NCHUNK = K1_EDGES_PER_TILE // K1_CHUNK
K1_FIRE = 3
K2_EDGES_PER_TILE = N_EDGES // 32
K2_CHUNK = 80
K2_NCHUNK = K2_EDGES_PER_TILE // K2_CHUNK
NBLK = 160

_MESH = plsc.VectorSubcoreMesh(core_axis_name="c", subcore_axis_name="s")
_PARAMS = pltpu.CompilerParams(needs_layout_passes=False,
                               use_tc_tiling_on_sc=False)


def _iota16():
    return lax.iota(jnp.int32, 16)


def _rsqrt_newton(d):
    xhalf = 0.5 * d
    i = lax.bitcast_convert_type(d, jnp.int32)
    i = jnp.int32(0x5F3759DF) - (i >> 1)
    y = lax.bitcast_convert_type(i, jnp.float32)
    for _ in range(3):
        y = y * (1.5 - xhalf * y * y)
    return jnp.where(d >= 0.5, y, 0.0)


def _scale_rows(buf, disbuf, nrows, dis_off):
    lane = _iota16()

    def body(i, _):
        dv = disbuf[pl.ds(dis_off + i * 16, 16)]
        for k in range(16):
            splat = jnp.sum(jnp.where(lane == k, dv, 0.0))
            n = i * 16 + k
            for j in range(H // 16):
                sl = pl.ds(j * 16, 16)
                buf[n, sl] = buf[n, sl] * splat
        return 0

    lax.fori_loop(0, nrows // 16, body, 0)


def _zero_buf(buf, nrows, ncols):
    zv = jnp.zeros((16,), jnp.float32)

    def body(i, _):
        for j in range(ncols // 16):
            buf[i, pl.ds(j * 16, 16)] = zv
        return 0

    lax.fori_loop(0, nrows, body, 0)


def _prepare_body(xp, pc, yout, disout, col_idx, nodebuf, disbuf, onesbuf,
                  deg):
    c = lax.axis_index("c")
    s = lax.axis_index("s")
    half_base = c * NPAD
    node_base = s * ROWS_PER_TILE

    pltpu.sync_copy(pc.at[s], col_idx)
    for j in range(K1_CHUNK // 16):
        onesbuf[pl.ds(j * 16, 16)] = jnp.ones((16,), jnp.float32)

    def zdis(i, _):
        disbuf[pl.ds(i * 16, 16)] = jnp.zeros((16,), jnp.float32)
        return 0

    lax.fori_loop(0, ROWS_PER_TILE // 16, zdis, 0)
    pltpu.sync_copy(disbuf, deg.at[pl.ds(node_base, ROWS_PER_TILE)])
    plsc.subcore_barrier()

    def deg_body(ch, _):
        pltpu.sync_copy(onesbuf, deg.at[col_idx.at[ch]], add=True)
        return 0

    lax.fori_loop(0, K1_NCHUNK, deg_body, 0)
    plsc.subcore_barrier()

    pltpu.sync_copy(deg.at[pl.ds(node_base, ROWS_PER_TILE)], disbuf)

    def dis_body(i, _):
        d = disbuf[pl.ds(i * 16, 16)]
        disbuf[pl.ds(i * 16, 16)] = _rsqrt_newton(d)
        return 0

    lax.fori_loop(0, ROWS_PER_TILE // 16, dis_body, 0)
    pltpu.sync_copy(disbuf, disout.at[pl.ds(half_base + node_base,
                                            ROWS_PER_TILE)])
    pltpu.sync_copy(xp.at[pl.ds(half_base + node_base, ROWS_PER_TILE)],
                    nodebuf)
    _scale_rows(nodebuf, disbuf, ROWS_PER_TILE, 0)
    pltpu.sync_copy(nodebuf, yout.at[pl.ds(half_base + node_base,
                                           ROWS_PER_TILE)])


@functools.partial(
    pl.kernel,
    out_type=(jax.ShapeDtypeStruct((2 * NPAD, H), jnp.float32),
              jax.ShapeDtypeStruct((2 * NPAD,), jnp.float32)),
    mesh=_MESH,
    compiler_params=_PARAMS,
    scratch_types=[
        pltpu.VMEM((K1_NCHUNK, K1_CHUNK), jnp.int32),
        pltpu.VMEM((ROWS_PER_TILE, H), jnp.float32),
        pltpu.VMEM((ROWS_PER_TILE,), jnp.float32),
        pltpu.VMEM((K1_CHUNK,), jnp.float32),
        pltpu.VMEM_SHARED((NPAD,), jnp.float32),
    ],
)
def _prepare(xp, pc, yout, disout, *rest):
    _prepare_body(xp, pc, yout, disout, *rest)


def _scatter_body(yin, dis, pr, pc, xagg, row_idx, col_idx, gbufs, nodebuf,
                  disbuf, z, sem_g, sem_g2, sem_s):
    c = lax.axis_index("c")
    s = lax.axis_index("s")
    half_base = c * NPAD
    node_base = s * ROWS_PER_TILE

    pltpu.sync_copy(pr.at[s], row_idx)
    pltpu.sync_copy(pc.at[s], col_idx)

    def shift_body(ch, _):
        for j in range(K1_CHUNK // 16):
            sl = pl.ds(j * 16, 16)
            row_idx[ch, sl] = row_idx[ch, sl] + half_base
        return 0

    lax.fori_loop(0, K1_NCHUNK, shift_body, 0)
    pltpu.sync_copy(dis.at[pl.ds(half_base + node_base, ROWS_PER_TILE)],
                    disbuf)

    _zero_buf(nodebuf, NBLK, H)
    for blk in range(ROWS_PER_TILE // NBLK):
        pltpu.sync_copy(nodebuf,
                        z.at[pl.ds(node_base + blk * NBLK, NBLK)])
    plsc.subcore_barrier()

    F = K1_FIRE
    sems = (sem_g, sem_g2)

    def fire_g(base, half):
        for k in range(F):
            pltpu.async_copy(yin.at[row_idx.at[base + k]],
                             gbufs[F * half + k], sems[half])

    def drain_g(base, half):
        for k in range(F):
            pltpu.make_async_copy(yin.at[row_idx.at[base + k]],
                                  gbufs[F * half + k], sems[half]).wait()

    def fire_s(base, half):
        for k in range(F):
            pltpu.async_copy(gbufs[F * half + k],
                             z.at[col_idx.at[base + k]], sem_s, add=True)

    def drain_s(base, half):
        for k in range(F):
            pltpu.make_async_copy(gbufs[F * half + k],
                                  z.at[col_idx.at[base + k]], sem_s).wait()

    n_super = K1_NCHUNK // (2 * F)
    fire_g(0, 0)

    def edge_body(i, _):
        b0 = i * 2 * F
        b1 = b0 + F
        drain_g(b0, 0)
        fire_g(b1, 1)
        fire_s(b0, 0)
        drain_s(b0, 0)
        drain_g(b1, 1)

        @pl.when(b1 + F < K1_NCHUNK)
        def _():
            fire_g(b1 + F, 0)

        fire_s(b1, 1)
        drain_s(b1, 1)
        return 0

    lax.fori_loop(0, n_super, edge_body, 0)
    rem = n_super * 2 * F
    drain_g(rem, 0)
    fire_s(rem, 0)
    drain_s(rem, 0)
    for ch in range(rem + F, K1_NCHUNK):
        pltpu.async_copy(yin.at[row_idx.at[ch]], gbufs[0], sem_g)
        pltpu.make_async_copy(yin.at[row_idx.at[ch]], gbufs[0], sem_g).wait()
        pltpu.sync_copy(gbufs[0], z.at[col_idx.at[ch]], add=True)
    plsc.subcore_barrier()

    for blk in range(ROWS_PER_TILE // NBLK):
        pltpu.sync_copy(z.at[pl.ds(node_base + blk * NBLK, NBLK)], nodebuf)
        _scale_rows(nodebuf, disbuf, NBLK, blk * NBLK)
        pltpu.sync_copy(nodebuf,
                        xagg.at[pl.ds(half_base + node_base + blk * NBLK,
                                      NBLK)])


@functools.partial(
    pl.kernel,
    out_type=jax.ShapeDtypeStruct((2 * NPAD, H), jnp.float32),
    mesh=_MESH,
    compiler_params=_PARAMS,
    scratch_types=[
        pltpu.VMEM((K1_NCHUNK, K1_CHUNK), jnp.int32),
        pltpu.VMEM((K1_NCHUNK, K1_CHUNK), jnp.int32),
        [pltpu.VMEM((K1_CHUNK, H), jnp.float32)] * (2 * K1_FIRE),
        pltpu.VMEM((NBLK, H), jnp.float32),
        pltpu.VMEM((ROWS_PER_TILE,), jnp.float32),
        pltpu.VMEM_SHARED((NPAD, H), jnp.float32),
        pltpu.SemaphoreType.DMA,
        pltpu.SemaphoreType.DMA,
        pltpu.SemaphoreType.DMA,
    ],
)
def _scatter(yin, dis, pr, pc, xagg, *rest):
    _scatter_body(yin, dis, pr, pc, xagg, *rest)


def _logits_body(xagg, slo, shi, dlo, dhi, out, silo, sihi, dilo, dihi,
                 bufs, lbuf, sem):
    c = lax.axis_index("c")
    s = lax.axis_index("s")
    w = c * 16 + s

    pltpu.sync_copy(slo.at[w], silo)
    pltpu.sync_copy(shi.at[w], sihi)
    pltpu.sync_copy(dlo.at[w], dilo)
    pltpu.sync_copy(dhi.at[w], dihi)

    idx_refs = (silo, sihi, dilo, dihi)

    def fire(ch, p):
        for k in range(4):
            pltpu.async_copy(xagg.at[idx_refs[k].at[ch]], bufs[4 * p + k],
                             sem)

    def drain(ch, p):
        for k in range(4):
            pltpu.make_async_copy(xagg.at[idx_refs[k].at[ch]],
                                  bufs[4 * p + k], sem).wait()

    def compute(ch, p):
        abuf0, abuf1, bbuf0, bbuf1 = bufs[4 * p:4 * p + 4]
        lane = _iota16()
        perms = [(lane + sh) % 16 for sh in (8, 4, 2, 1)]

        def group_body(g, _):
            base = g * 16
            ovec = jnp.zeros((16,), jnp.float32)
            for k in range(16):
                e = base + k
                acc = None
                for j in range(4):
                    sl = pl.ds(j * 16, 16)
                    t = abuf0[e, sl] * bbuf0[e, sl] + abuf1[e, sl] * bbuf1[e, sl]
                    acc = t if acc is None else acc + t
                for perm in perms:
                    acc = acc + jnp.take_along_axis(acc, perm, axis=0)
                ovec = jnp.where(lane == k, acc, ovec)
            lbuf[pl.ds(ch * K2_CHUNK + g * 16, 16)] = ovec
            return 0

        lax.fori_loop(0, K2_CHUNK // 16, group_body, 0)

    fire(0, 0)

    def pair_body(i, _):
        ch0 = i * 2
        drain(ch0, 0)
        fire(ch0 + 1, 1)
        compute(ch0, 0)
        drain(ch0 + 1, 1)

        @pl.when(ch0 + 2 < K2_NCHUNK)
        def _():
            fire(ch0 + 2, 0)

        compute(ch0 + 1, 1)
        return 0

    lax.fori_loop(0, K2_NCHUNK // 2, pair_body, 0)
    last = K2_NCHUNK - 1
    drain(last, 0)
    compute(last, 0)
    pltpu.sync_copy(lbuf, out.at[pl.ds(w * K2_EDGES_PER_TILE,
                                       K2_EDGES_PER_TILE)])


@functools.partial(
    pl.kernel,
    out_type=jax.ShapeDtypeStruct((N_EDGES,), jnp.float32),
    mesh=_MESH,
    compiler_params=_PARAMS,
    scratch_types=[
        pltpu.VMEM((K2_NCHUNK, K2_CHUNK), jnp.int32),
        pltpu.VMEM((K2_NCHUNK, K2_CHUNK), jnp.int32),
        pltpu.VMEM((K2_NCHUNK, K2_CHUNK), jnp.int32),
        pltpu.VMEM((K2_NCHUNK, K2_CHUNK), jnp.int32),
        [pltpu.VMEM((K2_CHUNK, H), jnp.float32)] * 8,
        pltpu.VMEM((K2_EDGES_PER_TILE,), jnp.float32),
        pltpu.SemaphoreType.DMA,
    ],
)
def _logits(xagg, slo, shi, dlo, dhi, out, *rest):
    _logits_body(xagg, slo, shi, dlo, dhi, out, *rest)


def kernel(x_input, W, b, edge_index_input, pos_edge_index):
    del W, b
    xp = jnp.zeros((2 * NPAD, H), jnp.float32)
    xp = xp.at[:N_NODES].set(x_input[:, :H])
    xp = xp.at[NPAD:NPAD + N_NODES].set(x_input[:, H:])

    pos = pos_edge_index.astype(jnp.int32)
    pr = pos[0].reshape(16, K1_NCHUNK, K1_CHUNK)
    pc = pos[1].reshape(16, K1_NCHUNK, K1_CHUNK)

    ei = edge_index_input.astype(jnp.int32)
    slo = ei[0].reshape(32, K2_NCHUNK, K2_CHUNK)
    shi = slo + NPAD
    dlo = ei[1].reshape(32, K2_NCHUNK, K2_CHUNK)
    dhi = dlo + NPAD

    y, dis = _prepare(xp, pc)
    xagg = _scatter(y, dis, pr, pc)
    return _logits(xagg, slo, shi, dlo, dhi)


# speedup vs baseline: 1.0558x; 1.0558x over previous
import functools

import jax
import jax.numpy as jnp
from jax import lax
from jax.experimental import pallas as pl
from jax.experimental.pallas import tpu as pltpu
from jax.experimental.pallas import tpu_sc as plsc

N_NODES = 10000
D = 128
H = 64
N_EDGES = 320000
NPAD = 10240
ROWS_PER_TILE = NPAD // 16
K1_EDGES_PER_TILE = N_EDGES // 16
K1_CHUNK = 80
K1_NCHUNK = K1_EDGES_PER_TILE // K1_CHUNK
K1_FIRE = 3
K2_EDGES_PER_TILE = N_EDGES // 32
K2_CHUNK = 80
K2_NCHUNK = K2_EDGES_PER_TILE // K2_CHUNK
NBLK = 160

_MESH = plsc.VectorSubcoreMesh(core_axis_name="c", subcore_axis_name="s")
_PARAMS = pltpu.CompilerParams(needs_layout_passes=False,
                               use_tc_tiling_on_sc=False)


def _iota16():
    return lax.iota(jnp.int32, 16)


def _rsqrt_newton(d):
    xhalf = 0.5 * d
    i = lax.bitcast_convert_type(d, jnp.int32)
    i = jnp.int32(0x5F3759DF) - (i >> 1)
    y = lax.bitcast_convert_type(i, jnp.float32)
    for _ in range(3):
        y = y * (1.5 - xhalf * y * y)
    return jnp.where(d >= 0.5, y, 0.0)


def _scale_rows(buf, disbuf, nrows, dis_off):
    lane = _iota16()

    def body(i, _):
        dv = disbuf[pl.ds(dis_off + i * 16, 16)]
        for k in range(16):
            splat = jnp.sum(jnp.where(lane == k, dv, 0.0))
            n = i * 16 + k
            for j in range(H // 16):
                sl = pl.ds(j * 16, 16)
                buf[n, sl] = buf[n, sl] * splat
        return 0

    lax.fori_loop(0, nrows // 16, body, 0)


def _zero_buf(buf, nrows, ncols):
    zv = jnp.zeros((16,), jnp.float32)

    def body(i, _):
        for j in range(ncols // 16):
            buf[i, pl.ds(j * 16, 16)] = zv
        return 0

    lax.fori_loop(0, nrows, body, 0)


def _prepare_body(xp, pc, yout, disout, col_idx, nodebuf, disbuf, onesbuf,
                  deg):
    c = lax.axis_index("c")
    s = lax.axis_index("s")
    half_base = c * NPAD
    node_base = s * ROWS_PER_TILE

    pltpu.sync_copy(pc.at[s], col_idx)
    for j in range(K1_CHUNK // 16):
        onesbuf[pl.ds(j * 16, 16)] = jnp.ones((16,), jnp.float32)

    def zdis(i, _):
        disbuf[pl.ds(i * 16, 16)] = jnp.zeros((16,), jnp.float32)
        return 0

    lax.fori_loop(0, ROWS_PER_TILE // 16, zdis, 0)
    pltpu.sync_copy(disbuf, deg.at[pl.ds(node_base, ROWS_PER_TILE)])
    plsc.subcore_barrier()

    def deg_body(ch, _):
        pltpu.sync_copy(onesbuf, deg.at[col_idx.at[ch]], add=True)
        return 0

    lax.fori_loop(0, K1_NCHUNK, deg_body, 0)
    plsc.subcore_barrier()

    pltpu.sync_copy(deg.at[pl.ds(node_base, ROWS_PER_TILE)], disbuf)

    def dis_body(i, _):
        d = disbuf[pl.ds(i * 16, 16)]
        disbuf[pl.ds(i * 16, 16)] = _rsqrt_newton(d)
        return 0

    lax.fori_loop(0, ROWS_PER_TILE // 16, dis_body, 0)
    pltpu.sync_copy(disbuf, disout.at[pl.ds(half_base + node_base,
                                            ROWS_PER_TILE)])
    pltpu.sync_copy(xp.at[pl.ds(half_base + node_base, ROWS_PER_TILE)],
                    nodebuf)
    _scale_rows(nodebuf, disbuf, ROWS_PER_TILE, 0)
    pltpu.sync_copy(nodebuf, yout.at[pl.ds(half_base + node_base,
                                           ROWS_PER_TILE)])


@functools.partial(
    pl.kernel,
    out_type=(jax.ShapeDtypeStruct((2 * NPAD, H), jnp.float32),
              jax.ShapeDtypeStruct((2 * NPAD,), jnp.float32)),
    mesh=_MESH,
    compiler_params=_PARAMS,
    scratch_types=[
        pltpu.VMEM((K1_NCHUNK, K1_CHUNK), jnp.int32),
        pltpu.VMEM((ROWS_PER_TILE, H), jnp.float32),
        pltpu.VMEM((ROWS_PER_TILE,), jnp.float32),
        pltpu.VMEM((K1_CHUNK,), jnp.float32),
        pltpu.VMEM_SHARED((NPAD,), jnp.float32),
    ],
)
def _prepare(xp, pc, yout, disout, *rest):
    _prepare_body(xp, pc, yout, disout, *rest)


def _scatter_body(yin, dis, pr, pc, xagg, row_idx, col_idx, gbufs, nodebuf,
                  disbuf, z, sem_g, sem_g2, sem_s):
    c = lax.axis_index("c")
    s = lax.axis_index("s")
    half_base = c * NPAD
    node_base = s * ROWS_PER_TILE

    pltpu.sync_copy(pr.at[s], row_idx)
    pltpu.sync_copy(pc.at[s], col_idx)

    def shift_body(ch, _):
        for j in range(K1_CHUNK // 16):
            sl = pl.ds(j * 16, 16)
            row_idx[ch, sl] = row_idx[ch, sl] + half_base
        return 0

    lax.fori_loop(0, K1_NCHUNK, shift_body, 0)
    pltpu.sync_copy(dis.at[pl.ds(half_base + node_base, ROWS_PER_TILE)],
                    disbuf)

    _zero_buf(nodebuf, NBLK, H)
    for blk in range(ROWS_PER_TILE // NBLK):
        pltpu.sync_copy(nodebuf,
                        z.at[pl.ds(node_base + blk * NBLK, NBLK)])
    plsc.subcore_barrier()

    F = K1_FIRE
    sems = (sem_g, sem_g2)

    def fire_g(base, half):
        for k in range(F):
            pltpu.async_copy(yin.at[row_idx.at[base + k]],
                             gbufs[F * half + k], sems[half])

    def drain_g(base, half):
        for k in range(F):
            pltpu.make_async_copy(yin.at[row_idx.at[base + k]],
                                  gbufs[F * half + k], sems[half]).wait()

    def fire_s(base, half):
        for k in range(F):
            pltpu.async_copy(gbufs[F * half + k],
                             z.at[col_idx.at[base + k]], sem_s, add=True)

    def drain_s(base, half):
        for k in range(F):
            pltpu.make_async_copy(gbufs[F * half + k],
                                  z.at[col_idx.at[base + k]], sem_s).wait()

    n_super = K1_NCHUNK // (2 * F)
    fire_g(0, 0)

    def edge_body(i, _):
        b0 = i * 2 * F
        b1 = b0 + F
        drain_g(b0, 0)
        fire_g(b1, 1)
        fire_s(b0, 0)
        drain_s(b0, 0)
        drain_g(b1, 1)

        @pl.when(b1 + F < K1_NCHUNK)
        def _():
            fire_g(b1 + F, 0)

        fire_s(b1, 1)
        drain_s(b1, 1)
        return 0

    lax.fori_loop(0, n_super, edge_body, 0)
    rem = n_super * 2 * F
    drain_g(rem, 0)
    fire_s(rem, 0)
    drain_s(rem, 0)
    for ch in range(rem + F, K1_NCHUNK):
        pltpu.async_copy(yin.at[row_idx.at[ch]], gbufs[0], sem_g)
        pltpu.make_async_copy(yin.at[row_idx.at[ch]], gbufs[0], sem_g).wait()
        pltpu.sync_copy(gbufs[0], z.at[col_idx.at[ch]], add=True)
    plsc.subcore_barrier()

    for blk in range(ROWS_PER_TILE // NBLK):
        pltpu.sync_copy(z.at[pl.ds(node_base + blk * NBLK, NBLK)], nodebuf)
        _scale_rows(nodebuf, disbuf, NBLK, blk * NBLK)
        pltpu.sync_copy(nodebuf,
                        xagg.at[pl.ds(half_base + node_base + blk * NBLK,
                                      NBLK)])


@functools.partial(
    pl.kernel,
    out_type=jax.ShapeDtypeStruct((2 * NPAD, H), jnp.float32),
    mesh=_MESH,
    compiler_params=_PARAMS,
    scratch_types=[
        pltpu.VMEM((K1_NCHUNK, K1_CHUNK), jnp.int32),
        pltpu.VMEM((K1_NCHUNK, K1_CHUNK), jnp.int32),
        [pltpu.VMEM((K1_CHUNK, H), jnp.float32)] * (2 * K1_FIRE),
        pltpu.VMEM((NBLK, H), jnp.float32),
        pltpu.VMEM((ROWS_PER_TILE,), jnp.float32),
        pltpu.VMEM_SHARED((NPAD, H), jnp.float32),
        pltpu.SemaphoreType.DMA,
        pltpu.SemaphoreType.DMA,
        pltpu.SemaphoreType.DMA,
    ],
)
def _scatter(yin, dis, pr, pc, xagg, *rest):
    _scatter_body(yin, dis, pr, pc, xagg, *rest)


def _logits_body(xagg, slo, shi, dlo, dhi, out, silo, sihi, dilo, dihi,
                 bufs, lbuf, sem):
    c = lax.axis_index("c")
    s = lax.axis_index("s")
    w = c * 16 + s

    pltpu.sync_copy(slo.at[w], silo)
    pltpu.sync_copy(shi.at[w], sihi)
    pltpu.sync_copy(dlo.at[w], dilo)
    pltpu.sync_copy(dhi.at[w], dihi)

    idx_refs = (silo, sihi, dilo, dihi)

    def fire(ch, p):
        for k in range(4):
            pltpu.async_copy(xagg.at[idx_refs[k].at[ch]], bufs[4 * p + k],
                             sem)

    def drain(ch, p):
        for k in range(4):
            pltpu.make_async_copy(xagg.at[idx_refs[k].at[ch]],
                                  bufs[4 * p + k], sem).wait()

    def compute(ch, p):
        abuf0, abuf1, bbuf0, bbuf1 = bufs[4 * p:4 * p + 4]
        lane = _iota16()

        def group_body(g, _):
            base = g * 16
            ovec = jnp.zeros((16,), jnp.float32)
            for k in range(16):
                e = base + k
                acc = None
                for j in range(4):
                    sl = pl.ds(j * 16, 16)
                    t = abuf0[e, sl] * bbuf0[e, sl] + abuf1[e, sl] * bbuf1[e, sl]
                    acc = t if acc is None else acc + t
                s = jnp.sum(acc)
                ovec = jnp.where(lane == k, s, ovec)
            lbuf[pl.ds(ch * K2_CHUNK + g * 16, 16)] = ovec
            return 0

        lax.fori_loop(0, K2_CHUNK // 16, group_body, 0)

    fire(0, 0)

    def pair_body(i, _):
        ch0 = i * 2
        drain(ch0, 0)
        fire(ch0 + 1, 1)
        compute(ch0, 0)
        drain(ch0 + 1, 1)

        @pl.when(ch0 + 2 < K2_NCHUNK)
        def _():
            fire(ch0 + 2, 0)

        compute(ch0 + 1, 1)
        return 0

    lax.fori_loop(0, K2_NCHUNK // 2, pair_body, 0)
    last = K2_NCHUNK - 1
    drain(last, 0)
    compute(last, 0)
    pltpu.sync_copy(lbuf, out.at[pl.ds(w * K2_EDGES_PER_TILE,
                                       K2_EDGES_PER_TILE)])


@functools.partial(
    pl.kernel,
    out_type=jax.ShapeDtypeStruct((N_EDGES,), jnp.float32),
    mesh=_MESH,
    compiler_params=_PARAMS,
    scratch_types=[
        pltpu.VMEM((K2_NCHUNK, K2_CHUNK), jnp.int32),
        pltpu.VMEM((K2_NCHUNK, K2_CHUNK), jnp.int32),
        pltpu.VMEM((K2_NCHUNK, K2_CHUNK), jnp.int32),
        pltpu.VMEM((K2_NCHUNK, K2_CHUNK), jnp.int32),
        [pltpu.VMEM((K2_CHUNK, H), jnp.float32)] * 8,
        pltpu.VMEM((K2_EDGES_PER_TILE,), jnp.float32),
        pltpu.SemaphoreType.DMA,
    ],
)
def _logits(xagg, slo, shi, dlo, dhi, out, *rest):
    _logits_body(xagg, slo, shi, dlo, dhi, out, *rest)


def kernel(x_input, W, b, edge_index_input, pos_edge_index):
    del W, b
    xp = jnp.zeros((2 * NPAD, H), jnp.float32)
    xp = xp.at[:N_NODES].set(x_input[:, :H])
    xp = xp.at[NPAD:NPAD + N_NODES].set(x_input[:, H:])

    pos = pos_edge_index.astype(jnp.int32)
    pr = pos[0].reshape(16, K1_NCHUNK, K1_CHUNK)
    pc = pos[1].reshape(16, K1_NCHUNK, K1_CHUNK)

    ei = edge_index_input.astype(jnp.int32)
    slo = ei[0].reshape(32, K2_NCHUNK, K2_CHUNK)
    shi = slo + NPAD
    dlo = ei[1].reshape(32, K2_NCHUNK, K2_CHUNK)
    dhi = dlo + NPAD

    y, dis = _prepare(xp, pc)
    xagg = _scatter(y, dis, pr, pc)
    return _logits(xagg, slo, shi, dlo, dhi)
